# Initial kernel scaffold; baseline (speedup 1.0000x reference)
import jax
import jax.numpy as jnp
from jax.experimental import pallas as pl


def kernel(user, item, genre, tag, W_wide, b_wide, user_table, item_table, W0, b0, W1, b1, Wf, bf):
    raise NotImplementedError("write your pallas kernel here")


# speedup vs baseline: 1.1860x; 1.1860x over previous
import jax
import jax.numpy as jnp
from jax import lax
from jax.experimental import pallas as pl
from jax.experimental.pallas import tpu as pltpu
from jax.experimental.pallas import tpu_sc as plsc

_NUM_USERS = 1000
_D = 128
_B = 4096
_NW = 32
_BPW = _B // _NW


def _sc_gather_body(user_hbm, item_hbm, emb2_hbm, ww_hbm,
                    gu_out, gi_out, wu_out, wi_out,
                    uidx, iidx, gu_v, gi_v, wu_v, wi_v, sem):
    wid = lax.axis_index("s") * 2 + lax.axis_index("c")
    base = wid * _BPW
    pltpu.sync_copy(user_hbm.at[pl.ds(base, _BPW)], uidx)
    pltpu.sync_copy(item_hbm.at[pl.ds(base, _BPW)], iidx)
    for j in range(_BPW // 16):
        iidx[pl.ds(j * 16, 16)] = iidx[pl.ds(j * 16, 16)] + _NUM_USERS
    c0 = pltpu.async_copy(emb2_hbm.at[uidx], gu_v, sem)
    c1 = pltpu.async_copy(emb2_hbm.at[iidx], gi_v, sem)
    c2 = pltpu.async_copy(ww_hbm.at[uidx], wu_v, sem)
    c3 = pltpu.async_copy(ww_hbm.at[iidx], wi_v, sem)
    c0.wait()
    c1.wait()
    c2.wait()
    c3.wait()
    pltpu.sync_copy(gu_v, gu_out.at[pl.ds(base, _BPW)])
    pltpu.sync_copy(gi_v, gi_out.at[pl.ds(base, _BPW)])
    pltpu.sync_copy(wu_v, wu_out.at[pl.ds(base, _BPW)])
    pltpu.sync_copy(wi_v, wi_out.at[pl.ds(base, _BPW)])


def _sc_gather(user, item, emb2, W_wide):
    mesh = plsc.VectorSubcoreMesh(core_axis_name="c", subcore_axis_name="s")
    f = pl.kernel(
        _sc_gather_body, mesh=mesh,
        out_type=tuple(
            jax.ShapeDtypeStruct((_B, _D), jnp.float32) for _ in range(4)),
        scratch_types=[
            pltpu.VMEM((_BPW,), jnp.int32),
            pltpu.VMEM((_BPW,), jnp.int32),
            pltpu.VMEM((_BPW, _D), jnp.float32),
            pltpu.VMEM((_BPW, _D), jnp.float32),
            pltpu.VMEM((_BPW, _D), jnp.float32),
            pltpu.VMEM((_BPW, _D), jnp.float32),
            pltpu.SemaphoreType.DMA,
        ],
    )
    return f(user, item, emb2, W_wide)


def _mlp_body(gu, gi, g, t, wu, wi, W0, b0, W1, b1, Wf, bf, b_wide, out):
    emb = gu[...] + gi[...]
    x = (jnp.dot(emb, W0[0:128, :], preferred_element_type=jnp.float32)
         + jnp.dot(g[...], W0[128:148, :], preferred_element_type=jnp.float32)
         + jnp.dot(t[...], W0[148:248, :], preferred_element_type=jnp.float32)
         + b0[...])
    h0 = jnp.maximum(x, 0.0)
    h1 = jnp.maximum(
        jnp.dot(h0, W1[...], preferred_element_type=jnp.float32) + b1[...], 0.0)
    wide = wu[...] + wi[...] + b_wide[...]
    logits = (jnp.dot(h1, Wf[0:128, :], preferred_element_type=jnp.float32)
              + jnp.dot(wide, Wf[128:256, :], preferred_element_type=jnp.float32)
              + bf[...])
    out[...] = logits


def _mlp(gu, gi, genre, tag, wu, wi, W0, b0, W1, b1, Wf, bf, b_wide):
    nb = 4
    blk = _B // nb
    rep = lambda shape: pl.BlockSpec(shape, lambda i: (0,) * len(shape))
    row = lambda d: pl.BlockSpec((blk, d), lambda i: (i, 0))
    return pl.pallas_call(
        _mlp_body,
        grid=(nb,),
        in_specs=[
            row(_D), row(_D), row(20), row(100), row(_D), row(_D),
            rep((248, 256)), rep((256,)), rep((256, 128)), rep((128,)),
            rep((256, 1)), rep((1,)), rep((128,)),
        ],
        out_specs=row(1),
        out_shape=jax.ShapeDtypeStruct((_B, 1), jnp.float32),
    )(gu, gi, genre, tag, wu, wi, W0, b0, W1, b1, Wf, bf, b_wide)


def kernel(user, item, genre, tag, W_wide, b_wide, user_table, item_table,
           W0, b0, W1, b1, Wf, bf):
    user = user.astype(jnp.int32)
    item = item.astype(jnp.int32)
    zeros = jnp.zeros_like(user_table)
    emb2 = jnp.concatenate([
        jnp.concatenate([user_table, zeros], axis=1),
        jnp.concatenate([zeros, item_table], axis=1),
    ], axis=0)
    gu, gi, wu, wi = _sc_gather(user, item, emb2, W_wide)
    return _mlp(gu, gi, genre, tag, wu, wi, W0, b0, W1, b1, Wf, bf, b_wide)
